# baseline (speedup 1.0000x reference)
import jax
import jax.numpy as jnp
from jax.experimental import pallas as pl

_TILE = 256


def _gat_dense_kernel(x_ref, w_ref, asrc_ref, adst_ref, h_ref, al_ref, ar_ref):
    h = jnp.dot(x_ref[...], w_ref[...], preferred_element_type=jnp.float32)
    h_ref[...] = h
    heads, c = asrc_ref.shape
    h3 = h.reshape(h.shape[0], heads, c)
    al_ref[...] = jnp.sum(h3 * asrc_ref[...][None, :, :], axis=-1)
    ar_ref[...] = jnp.sum(h3 * adst_ref[...][None, :, :], axis=-1)


def _dense_stage(x, W, a_src, a_dst):
    n, din = x.shape
    hd = W.shape[1]
    heads, c = a_src.shape
    npad = ((n + _TILE - 1) // _TILE) * _TILE
    xp = jnp.pad(x, ((0, npad - n), (0, 0)))
    h, al, ar = pl.pallas_call(
        _gat_dense_kernel,
        grid=(npad // _TILE,),
        in_specs=[
            pl.BlockSpec((_TILE, din), lambda i: (i, 0)),
            pl.BlockSpec((din, hd), lambda i: (0, 0)),
            pl.BlockSpec((heads, c), lambda i: (0, 0)),
            pl.BlockSpec((heads, c), lambda i: (0, 0)),
        ],
        out_specs=[
            pl.BlockSpec((_TILE, hd), lambda i: (i, 0)),
            pl.BlockSpec((_TILE, heads), lambda i: (i, 0)),
            pl.BlockSpec((_TILE, heads), lambda i: (i, 0)),
        ],
        out_shape=[
            jax.ShapeDtypeStruct((npad, hd), jnp.float32),
            jax.ShapeDtypeStruct((npad, heads), jnp.float32),
            jax.ShapeDtypeStruct((npad, heads), jnp.float32),
        ],
    )(xp, W, a_src, a_dst)
    return h[:n].reshape(n, heads, c), al[:n], ar[:n]


def _gat_layer(x, src, dst, W, a_src, a_dst, b):
    n = x.shape[0]
    h, al, ar = _dense_stage(x, W, a_src, a_dst)
    e = al[src] + ar[dst]
    e = jnp.where(e > 0, e, 0.2 * e)
    emax = jax.ops.segment_max(e, dst, num_segments=n)
    emax = jnp.where(jnp.isfinite(emax), emax, 0.0)
    ee = jnp.exp(e - emax[dst])
    den = jax.ops.segment_sum(ee, dst, num_segments=n)
    alpha = ee / (den[dst] + 1e-16)
    msg = h[src] * alpha[:, :, None]
    out = jax.ops.segment_sum(msg, dst, num_segments=n)
    return out.mean(axis=1) + b


def kernel(x, edge_index, batch, W0, a_src0, a_dst0, b0,
           W1, a_src1, a_dst1, b1, W2, a_src2, a_dst2, b2):
    src, dst = edge_index[0], edge_index[1]
    h = jax.nn.relu(_gat_layer(x, src, dst, W0, a_src0, a_dst0, b0))
    h = jax.nn.relu(_gat_layer(h, src, dst, W1, a_src1, a_dst1, b1))
    h = jax.nn.relu(_gat_layer(h, src, dst, W2, a_src2, a_dst2, b2))
    return h
